# baseline (speedup 1.0000x reference)
import functools

import jax
import jax.numpy as jnp
from jax import lax
from jax.experimental import pallas as pl
from jax.experimental.pallas import tpu as pltpu
from jax.experimental.pallas import tpu_sc as plsc

BATCH = 16384
EMBED = 32
HID = 64
ACT = 6

NUM_CORES = 2
NUM_SUBCORES = 16
NUM_WORKERS = NUM_CORES * NUM_SUBCORES
B_PER_W = BATCH // NUM_WORKERS


def _sc_gather(table, idx):
    mesh = plsc.VectorSubcoreMesh(core_axis_name="c", subcore_axis_name="s")

    @functools.partial(
        pl.kernel,
        mesh=mesh,
        out_type=jax.ShapeDtypeStruct((BATCH, EMBED), jnp.float32),
        scratch_types=[
            pltpu.VMEM((B_PER_W,), jnp.int32),
            pltpu.VMEM((B_PER_W, EMBED), jnp.float32),
            pltpu.SemaphoreType.DMA,
            pltpu.SemaphoreType.DMA,
            pltpu.SemaphoreType.DMA,
            pltpu.SemaphoreType.DMA,
            pltpu.SemaphoreType.DMA,
            pltpu.SemaphoreType.DMA,
            pltpu.SemaphoreType.DMA,
            pltpu.SemaphoreType.DMA,
            pltpu.SemaphoreType.DMA,
        ],
        compiler_params=pltpu.CompilerParams(use_tc_tiling_on_sc=False),
    )
    def gather_kernel(idx_hbm, table_hbm, out_hbm, idx_v, rows_v, sem_i,
                      s0, s1, s2, s3, s4, s5, s6, s7):
        sems = (s0, s1, s2, s3, s4, s5, s6, s7)
        wid = lax.axis_index("s") * NUM_CORES + lax.axis_index("c")
        base = wid * B_PER_W
        pltpu.async_copy(idx_hbm.at[pl.ds(base, B_PER_W)], idx_v, sem_i).wait()

        @pl.loop(0, B_PER_W, step=16)
        def _(i):
            vec = idx_v[pl.ds(i, 16)]
            for j in range(16):
                pltpu.async_copy(
                    table_hbm.at[pl.ds(vec[j], 1)],
                    rows_v.at[pl.ds(i + j, 1)],
                    sems[j % 8],
                )

        for q in range(8):
            pltpu.make_async_copy(
                table_hbm.at[pl.ds(0, B_PER_W // 8)],
                rows_v.at[pl.ds(q * (B_PER_W // 8), B_PER_W // 8)],
                sems[q],
            ).wait()
        pltpu.async_copy(rows_v, out_hbm.at[pl.ds(base, B_PER_W)], sem_i).wait()

    return gather_kernel(idx, table)


def _mlp_body(x_ref, w1_ref, b1_ref, w2_ref, b2_ref, o_ref):
    h = jnp.dot(x_ref[...], w1_ref[...], preferred_element_type=jnp.float32)
    h = jnp.maximum(h + b1_ref[...], 0.0)
    o = jnp.dot(h, w2_ref[...], preferred_element_type=jnp.float32)
    o_ref[...] = o + b2_ref[...]


def _tc_mlp(x, W1, b1, W2, b2):
    nblk = 8
    blk = BATCH // nblk
    return pl.pallas_call(
        _mlp_body,
        grid=(nblk,),
        in_specs=[
            pl.BlockSpec((blk, EMBED), lambda i: (i, 0)),
            pl.BlockSpec((EMBED, HID), lambda i: (0, 0)),
            pl.BlockSpec((1, HID), lambda i: (0, 0)),
            pl.BlockSpec((HID, ACT), lambda i: (0, 0)),
            pl.BlockSpec((1, ACT), lambda i: (0, 0)),
        ],
        out_specs=pl.BlockSpec((blk, ACT), lambda i: (i, 0)),
        out_shape=jax.ShapeDtypeStruct((BATCH, ACT), jnp.float32),
    )(x, W1, b1.reshape(1, HID), W2, b2.reshape(1, ACT))


def kernel(state, table, W1, b1, W2, b2):
    x = _sc_gather(table, state.astype(jnp.int32))
    return _tc_mlp(x, W1, b1, W2, b2)
